# Initial kernel scaffold; baseline (speedup 1.0000x reference)
import jax
import jax.numpy as jnp
from jax.experimental import pallas as pl


def kernel(logits, labels):
    raise NotImplementedError("write your pallas kernel here")


# speedup vs baseline: 1.0393x; 1.0393x over previous
import functools
import math

import jax
import jax.numpy as jnp
from jax.experimental import pallas as pl

S = 64.0
M1 = 0.6
M2 = 0.4

_R = 256
_C = 2048


def _asin01(x):
    z_small = x * x
    w = 0.5 * (1.0 - x)
    s = jnp.sqrt(w)
    small = x < 0.5
    z = jnp.where(small, z_small, w)
    p = z * (0.16666666666666666 + z * (-0.3255658186224009 + z * (
        0.20121253213486293 + z * (-0.04005553450067941 + z * (
            7.915349942898145e-4 + z * 3.4793310759602117e-5)))))
    q = 1.0 + z * (-2.403394911734414 + z * (2.0209457602335057 + z * (
        -0.688283971605453 + z * 0.07703815055590194)))
    r = p / q
    u = jnp.where(small, x, s)
    asin_u = u + u * r
    return jnp.where(small, asin_u, 0.5 * math.pi - 2.0 * asin_u)


def _tan_small(y):
    y2 = y * y
    sin_y = y * (1.0 + y2 * (-1.0 / 6.0 + y2 * (1.0 / 120.0 + y2 * (
        -1.0 / 5040.0 + y2 * (1.0 / 362880.0)))))
    cos_y = 1.0 + y2 * (-0.5 + y2 * (1.0 / 24.0 + y2 * (-1.0 / 720.0 + y2 * (
        1.0 / 40320.0 + y2 * (-1.0 / 3628800.0)))))
    return sin_y / cos_y


def _tanface_block(labels_ref, x_ref, out_ref):
    j = pl.program_id(1)
    lab = labels_ref[0, 0, :]
    valid = lab >= 0
    rel = jnp.where(valid, lab, -1) - j * _C
    x = x_ref[...]
    col = jax.lax.broadcasted_iota(jnp.int32, (_R, _C), 1)
    match = col == rel[:, None]
    target = jnp.sum(jnp.where(match, x, 0.0), axis=1)
    newv = _tan_small(M1 * _asin01(target)) - M2
    out_ref[...] = jnp.where(match, newv[:, None], x) * S


@jax.jit
def kernel(logits, labels):
    B, V = logits.shape
    nr = B // _R
    nc = (V + _C - 1) // _C
    lab3 = labels.astype(jnp.int32).reshape(nr, 1, _R)
    return pl.pallas_call(
        _tanface_block,
        grid=(nr, nc),
        in_specs=[
            pl.BlockSpec((1, 1, _R), lambda i, j: (i, 0, 0)),
            pl.BlockSpec((_R, _C), lambda i, j: (i, j)),
        ],
        out_specs=pl.BlockSpec((_R, _C), lambda i, j: (i, j)),
        out_shape=jax.ShapeDtypeStruct((B, V), jnp.float32),
    )(lab3, logits)


# speedup vs baseline: 1.1152x; 1.0731x over previous
import math

import jax
import jax.numpy as jnp
from jax.experimental import pallas as pl
from jax.experimental.pallas import tpu as pltpu

S = 64.0
M1 = 0.6
M2 = 0.4

_RCH = 8
_NBUF = 8


def _asin01(x):
    z_small = x * x
    w = 0.5 * (1.0 - x)
    s = jnp.sqrt(w)
    small = x < 0.5
    z = jnp.where(small, z_small, w)
    p = z * (0.16666666666666666 + z * (-0.3255658186224009 + z * (
        0.20121253213486293 + z * (-0.04005553450067941 + z * (
            7.915349942898145e-4 + z * 3.4793310759602117e-5)))))
    q = 1.0 + z * (-2.403394911734414 + z * (2.0209457602335057 + z * (
        -0.688283971605453 + z * 0.07703815055590194)))
    r = p / q
    u = jnp.where(small, x, s)
    asin_u = u + u * r
    return jnp.where(small, asin_u, 0.5 * math.pi - 2.0 * asin_u)


def _tan_small(y):
    y2 = y * y
    sin_y = y * (1.0 + y2 * (-1.0 / 6.0 + y2 * (1.0 / 120.0 + y2 * (
        -1.0 / 5040.0 + y2 * (1.0 / 362880.0)))))
    cos_y = 1.0 + y2 * (-0.5 + y2 * (1.0 / 24.0 + y2 * (-1.0 / 720.0 + y2 * (
        1.0 / 40320.0 + y2 * (-1.0 / 3628800.0)))))
    return sin_y / cos_y


def _body(lab_ref, x_hbm, out_hbm, in_buf, out_buf, in_sems, out_sems):
    n_rows, v = x_hbm.shape
    nch = n_rows // _RCH

    def in_copy(i, b):
        return pltpu.make_async_copy(
            x_hbm.at[pl.ds(i * _RCH, _RCH)], in_buf.at[b], in_sems.at[b])

    def out_copy(i, b):
        return pltpu.make_async_copy(
            out_buf.at[b], out_hbm.at[pl.ds(i * _RCH, _RCH)], out_sems.at[b])

    for b in range(_NBUF):
        in_copy(b, b).start()

    def step(i, carry):
        b = jax.lax.rem(i, _NBUF)
        in_copy(i, b).wait()

        @pl.when(i >= _NBUF)
        def _():
            out_copy(i - _NBUF, b).wait()

        x = in_buf[b]
        lab = lab_ref[i]
        lab_s = jnp.where(lab >= 0, lab, -1)
        col = jax.lax.broadcasted_iota(jnp.int32, (_RCH, v), 1)
        match = col == lab_s[:, None]
        target = jnp.sum(jnp.where(match, x, 0.0), axis=1)
        newv = _tan_small(M1 * _asin01(target)) - M2
        out_buf[b] = jnp.where(match, newv[:, None], x) * S

        out_copy(i, b).start()

        @pl.when(i + _NBUF < nch)
        def _():
            in_copy(i + _NBUF, b).start()

        return carry

    jax.lax.fori_loop(0, nch, step, 0)

    def drain(i, carry):
        b = jax.lax.rem(i, _NBUF)
        out_copy(i, b).wait()
        return carry

    jax.lax.fori_loop(nch - _NBUF, nch, drain, 0)


@jax.jit
def kernel(logits, labels):
    B, V = logits.shape
    nch = B // _RCH
    lab2 = labels.astype(jnp.int32).reshape(nch, _RCH)
    return pl.pallas_call(
        _body,
        in_specs=[
            pl.BlockSpec(memory_space=pltpu.MemorySpace.VMEM),
            pl.BlockSpec(memory_space=pltpu.MemorySpace.HBM),
        ],
        out_specs=pl.BlockSpec(memory_space=pltpu.MemorySpace.HBM),
        out_shape=jax.ShapeDtypeStruct((B, V), jnp.float32),
        scratch_shapes=[
            pltpu.VMEM((_NBUF, _RCH, V), jnp.float32),
            pltpu.VMEM((_NBUF, _RCH, V), jnp.float32),
            pltpu.SemaphoreType.DMA((_NBUF,)),
            pltpu.SemaphoreType.DMA((_NBUF,)),
        ],
    )(lab2, logits)
